# Initial kernel scaffold; baseline (speedup 1.0000x reference)
import jax
import jax.numpy as jnp
from jax.experimental import pallas as pl


def kernel(points, W_d, b_d, W_a, b_a):
    raise NotImplementedError("write your pallas kernel here")


# speedup vs baseline: 1.3072x; 1.3072x over previous
import numpy as np
import jax
import jax.numpy as jnp
from jax import lax
from jax.experimental import pallas as pl

_N, _H = 384, 64
_SIGMA_D = 0.2
_SIGMA_A = 15.0
_K = 3
_FACTOR_A = 180.0 / (_SIGMA_A * np.pi)
_ROWS = 8


def _tc_kernel(pi_ref, ptsT_ref, div_ref, wd_ref, wa_ref, bd_ref, ba_ref,
               out_ref):
    R, N = _ROWS, _N
    px_i = pi_ref[:, 0:1]
    py_i = pi_ref[:, 1:2]
    pz_i = pi_ref[:, 2:3]
    px_j = ptsT_ref[0:1, :]
    py_j = ptsT_ref[1:2, :]
    pz_j = ptsT_ref[2:3, :]

    x2 = px_i * px_i + py_i * py_i + pz_i * pz_i
    y2 = px_j * px_j + py_j * py_j + pz_j * pz_j
    bx_i = px_i.astype(jnp.bfloat16).astype(jnp.float32)
    by_i = py_i.astype(jnp.bfloat16).astype(jnp.float32)
    bz_i = pz_i.astype(jnp.bfloat16).astype(jnp.float32)
    bx_j = px_j.astype(jnp.bfloat16).astype(jnp.float32)
    by_j = py_j.astype(jnp.bfloat16).astype(jnp.float32)
    bz_j = pz_j.astype(jnp.bfloat16).astype(jnp.float32)
    xy = bx_i * bx_j + by_i * by_j + bz_i * bz_j
    d2 = jnp.maximum(x2 - 2.0 * xy + y2, 0.0)
    dist = jnp.sqrt(d2)
    d_idx = dist / _SIGMA_D

    iota_j = lax.broadcasted_iota(jnp.int32, (R, N), 1)
    cur = dist
    masks = []
    for _ in range(_K + 1):
        mn = jnp.min(cur, axis=1, keepdims=True)
        cand = jnp.where(cur == mn, iota_j, N)
        idx = jnp.min(cand, axis=1, keepdims=True)
        m = iota_j == idx
        masks.append(m)
        cur = jnp.where(m, jnp.inf, cur)

    ax = px_j - px_i
    ay = py_j - py_i
    az = pz_j - pz_i

    a_rows = []
    for t in range(1, _K + 1):
        m = masks[t]
        nx = jnp.sum(jnp.where(m, px_j, 0.0), axis=1, keepdims=True)
        ny = jnp.sum(jnp.where(m, py_j, 0.0), axis=1, keepdims=True)
        nz = jnp.sum(jnp.where(m, pz_j, 0.0), axis=1, keepdims=True)
        rx = nx - px_i
        ry = ny - py_i
        rz = nz - pz_i
        cx = ry * az - rz * ay
        cy = rz * ax - rx * az
        cz = rx * ay - ry * ax
        sinv = jnp.sqrt(cx * cx + cy * cy + cz * cz)
        cosv = rx * ax + ry * ay + rz * az
        cosv = jnp.where(cosv == 0.0, 0.0, cosv)
        ang = jnp.arctan2(sinv, cosv)
        a_rows.append(ang * _FACTOR_A)

    div2 = div_ref[...]
    wd = wd_ref[...]
    wa = wa_ref[...]
    bd = bd_ref[...]
    ba = ba_ref[...]
    for r in range(R):
        om_d = div2 * d_idx[r:r + 1, :]
        emb_d = jnp.concatenate([jnp.sin(om_d), jnp.cos(om_d)], axis=0)
        embs_a = []
        for t in range(_K):
            om_a = div2 * a_rows[t][r:r + 1, :]
            embs_a.append(
                jnp.concatenate([jnp.sin(om_a), jnp.cos(om_a)], axis=0))
        emb_a = jnp.concatenate(embs_a, axis=1)
        d_res = jnp.dot(wd, emb_d.astype(jnp.bfloat16),
                        preferred_element_type=jnp.float32)
        a_res = jnp.dot(wa, emb_a.astype(jnp.bfloat16),
                        preferred_element_type=jnp.float32)
        a_max = jnp.maximum(jnp.maximum(a_res[:, :N], a_res[:, N:2 * N]),
                            a_res[:, 2 * N:])
        out_ref[r] = d_res + bd + a_max + ba


def kernel(points, W_d, b_d, W_a, b_a):
    N, H = _N, _H
    pts = points.reshape(N, 3)
    ptsT = pts.T
    div_indices = jnp.arange(0, H, 2, dtype=jnp.float32)
    div_term = jnp.exp(div_indices * (-np.log(10000.0) / H))
    div2 = div_term.reshape(32, 1)
    wd_cat = jnp.concatenate([W_d[:, 0::2], W_d[:, 1::2]],
                             axis=1).astype(jnp.bfloat16)
    wa_cat = jnp.concatenate([W_a[:, 0::2], W_a[:, 1::2]],
                             axis=1).astype(jnp.bfloat16)
    bd = b_d.reshape(H, 1)
    ba = b_a.reshape(H, 1)

    out = pl.pallas_call(
        _tc_kernel,
        grid=(N // _ROWS,),
        in_specs=[
            pl.BlockSpec((_ROWS, 3), lambda i: (i, 0)),
            pl.BlockSpec((3, N), lambda i: (0, 0)),
            pl.BlockSpec((32, 1), lambda i: (0, 0)),
            pl.BlockSpec((H, H), lambda i: (0, 0)),
            pl.BlockSpec((H, H), lambda i: (0, 0)),
            pl.BlockSpec((H, 1), lambda i: (0, 0)),
            pl.BlockSpec((H, 1), lambda i: (0, 0)),
        ],
        out_specs=pl.BlockSpec((_ROWS, H, N), lambda i: (i, 0, 0)),
        out_shape=jax.ShapeDtypeStruct((N, H, N), jnp.float32),
    )(pts, ptsT, div2, wd_cat, wa_cat, bd, ba)

    return jnp.transpose(out, (0, 2, 1))[None]


# speedup vs baseline: 2.4609x; 1.8826x over previous
import numpy as np
import jax
import jax.numpy as jnp
from jax import lax
from jax.experimental import pallas as pl

_N, _H = 384, 64
_SIGMA_D = 0.2
_SIGMA_A = 15.0
_K = 3
_FACTOR_A = 180.0 / (_SIGMA_A * np.pi)
_ROWS = 8

_TWO_OVER_PI = np.float32(2.0 / np.pi)
_PIO2_HI = np.float32(1.57080078125)
_PIO2_LO = np.float32(np.pi / 2 - 1.57080078125)
_S1, _S2, _S3 = np.float32(-1.6666667e-1), np.float32(8.3333337e-3), \
    np.float32(-1.9841270e-4)
_C1, _C2, _C3 = np.float32(-0.5), np.float32(4.1666668e-2), \
    np.float32(-1.3888889e-3)


def _sincos(x):
    k = jnp.floor(x * _TWO_OVER_PI + 0.5)
    ki = k.astype(jnp.int32)
    r = (x - k * _PIO2_HI) - k * _PIO2_LO
    r2 = r * r
    sp = r * (1.0 + r2 * (_S1 + r2 * (_S2 + r2 * _S3)))
    cp = 1.0 + r2 * (_C1 + r2 * (_C2 + r2 * _C3))
    swap = (ki & 1) == 1
    s = jnp.where(swap, cp, sp)
    c = jnp.where(swap, sp, cp)
    s = jnp.where((ki & 2) == 2, -s, s)
    c = jnp.where(((ki + 1) & 2) == 2, -c, c)
    return s, c


def _tc_kernel(pi_ref, ptsT_ref, div_ref, wd_ref, wa_ref, bd_ref, ba_ref,
               out_ref):
    R, N = _ROWS, _N
    px_i = pi_ref[:, 0:1]
    py_i = pi_ref[:, 1:2]
    pz_i = pi_ref[:, 2:3]
    px_j = ptsT_ref[0:1, :]
    py_j = ptsT_ref[1:2, :]
    pz_j = ptsT_ref[2:3, :]

    x2 = px_i * px_i + py_i * py_i + pz_i * pz_i
    y2 = px_j * px_j + py_j * py_j + pz_j * pz_j
    bx_i = px_i.astype(jnp.bfloat16).astype(jnp.float32)
    by_i = py_i.astype(jnp.bfloat16).astype(jnp.float32)
    bz_i = pz_i.astype(jnp.bfloat16).astype(jnp.float32)
    bx_j = px_j.astype(jnp.bfloat16).astype(jnp.float32)
    by_j = py_j.astype(jnp.bfloat16).astype(jnp.float32)
    bz_j = pz_j.astype(jnp.bfloat16).astype(jnp.float32)
    xy = bx_i * bx_j + by_i * by_j + bz_i * bz_j
    d2 = jnp.maximum(x2 - 2.0 * xy + y2, 0.0)
    dist = jnp.sqrt(d2)
    d_idx = dist / _SIGMA_D

    iota_j = lax.broadcasted_iota(jnp.int32, (R, N), 1)
    cur = dist
    masks = []
    for _ in range(_K + 1):
        mn = jnp.min(cur, axis=1, keepdims=True)
        cand = jnp.where(cur == mn, iota_j, N)
        idx = jnp.min(cand, axis=1, keepdims=True)
        m = iota_j == idx
        masks.append(m)
        cur = jnp.where(m, jnp.inf, cur)

    ax = px_j - px_i
    ay = py_j - py_i
    az = pz_j - pz_i

    a_rows = []
    for t in range(1, _K + 1):
        m = masks[t]
        nx = jnp.sum(jnp.where(m, px_j, 0.0), axis=1, keepdims=True)
        ny = jnp.sum(jnp.where(m, py_j, 0.0), axis=1, keepdims=True)
        nz = jnp.sum(jnp.where(m, pz_j, 0.0), axis=1, keepdims=True)
        rx = nx - px_i
        ry = ny - py_i
        rz = nz - pz_i
        cx = ry * az - rz * ay
        cy = rz * ax - rx * az
        cz = rx * ay - ry * ax
        sinv = jnp.sqrt(cx * cx + cy * cy + cz * cz)
        cosv = rx * ax + ry * ay + rz * az
        cosv = jnp.where(cosv == 0.0, 0.0, cosv)
        ang = jnp.arctan2(sinv, cosv)
        a_rows.append(ang * _FACTOR_A)

    div3 = div_ref[...]
    wd = wd_ref[...]
    wa = wa_ref[...]
    bd = bd_ref[...]
    ba = ba_ref[...]
    idx_all = jnp.concatenate([d_idx] + a_rows, axis=0)
    om3 = idx_all[:, None, :] * div3
    sp3, cp3 = _sincos(om3)
    for r in range(R):
        emb_d = jnp.concatenate([sp3[r], cp3[r]], axis=0)
        embs_a = []
        for t in range(1, _K + 1):
            embs_a.append(
                jnp.concatenate([sp3[t * R + r], cp3[t * R + r]], axis=0))
        emb_a = jnp.concatenate(embs_a, axis=1)
        d_res = jnp.dot(wd, emb_d.astype(jnp.bfloat16),
                        preferred_element_type=jnp.float32)
        a_res = jnp.dot(wa, emb_a.astype(jnp.bfloat16),
                        preferred_element_type=jnp.float32)
        a_max = jnp.maximum(jnp.maximum(a_res[:, :N], a_res[:, N:2 * N]),
                            a_res[:, 2 * N:])
        out_ref[r] = d_res + bd + a_max + ba


def kernel(points, W_d, b_d, W_a, b_a):
    N, H = _N, _H
    pts = points.reshape(N, 3)
    ptsT = pts.T
    div_indices = jnp.arange(0, H, 2, dtype=jnp.float32)
    div_term = jnp.exp(div_indices * (-np.log(10000.0) / H))
    div3 = div_term.reshape(1, 32, 1)
    wd_cat = jnp.concatenate([W_d[:, 0::2], W_d[:, 1::2]],
                             axis=1).astype(jnp.bfloat16)
    wa_cat = jnp.concatenate([W_a[:, 0::2], W_a[:, 1::2]],
                             axis=1).astype(jnp.bfloat16)
    bd = b_d.reshape(H, 1)
    ba = b_a.reshape(H, 1)

    out = pl.pallas_call(
        _tc_kernel,
        grid=(N // _ROWS,),
        in_specs=[
            pl.BlockSpec((_ROWS, 3), lambda i: (i, 0)),
            pl.BlockSpec((3, N), lambda i: (0, 0)),
            pl.BlockSpec((1, 32, 1), lambda i: (0, 0, 0)),
            pl.BlockSpec((H, H), lambda i: (0, 0)),
            pl.BlockSpec((H, H), lambda i: (0, 0)),
            pl.BlockSpec((H, 1), lambda i: (0, 0)),
            pl.BlockSpec((H, 1), lambda i: (0, 0)),
        ],
        out_specs=pl.BlockSpec((_ROWS, H, N), lambda i: (i, 0, 0)),
        out_shape=jax.ShapeDtypeStruct((N, H, N), jnp.float32),
    )(pts, ptsT, div3, wd_cat, wa_cat, bd, ba)

    return jnp.transpose(out, (0, 2, 1))[None]
